# baseline (speedup 1.0000x reference)
import functools

import jax
import jax.numpy as jnp
from jax import lax
from jax.experimental import pallas as pl
from jax.experimental.pallas import tpu as pltpu
from jax.experimental.pallas import tpu_sc as plsc

N_ROWS = 16384
N_COLS = 256
K_OUT = 128
NUM_WORKERS = 32
ROWS_PER_WORKER = N_ROWS // NUM_WORKERS
BLK_ROWS = 64
NBLK = ROWS_PER_WORKER // BLK_ROWS
NBUF_IN = 3
NBUF_OUT = 2
LANES = 16


def _build_sc_kernel():
    mesh = plsc.VectorSubcoreMesh(core_axis_name="c", subcore_axis_name="s")

    @functools.partial(
        pl.kernel,
        mesh=mesh,
        out_type=jax.ShapeDtypeStruct((N_ROWS, K_OUT), jnp.float32),
        compiler_params=pltpu.CompilerParams(
            needs_layout_passes=False,
            use_tc_tiling_on_sc=True,
        ),
        scratch_types=[
            pltpu.VMEM((NBUF_IN, BLK_ROWS, N_COLS), jnp.float32),
            pltpu.VMEM((NBUF_OUT, BLK_ROWS, K_OUT), jnp.float32),
            pltpu.SemaphoreType.DMA((NBUF_IN,)),
            pltpu.SemaphoreType.DMA((NBUF_OUT,)),
        ],
    )
    def k(in_hbm, out_hbm, in_v, out_v, in_sem, out_sem):
        wid = lax.axis_index("s") * 2 + lax.axis_index("c")
        lane2 = 2 * lax.iota(jnp.int32, LANES)
        cols = [lane2 + (2 * LANES * t) for t in range(K_OUT // LANES)]

        def row0(b):
            return wid * ROWS_PER_WORKER + b * BLK_ROWS

        def start_in(b):
            return pltpu.async_copy(
                in_hbm.at[pl.ds(row0(b), BLK_ROWS), :],
                in_v.at[b % NBUF_IN],
                in_sem.at[b % NBUF_IN],
            )

        def start_out(b):
            return pltpu.async_copy(
                out_v.at[b % NBUF_OUT],
                out_hbm.at[pl.ds(row0(b), BLK_ROWS), :],
                out_sem.at[b % NBUF_OUT],
            )

        in_copies = {b: start_in(b) for b in range(min(NBUF_IN, NBLK))}
        out_copies = {}
        for b in range(NBLK):
            in_copies.pop(b).wait()
            if b >= NBUF_OUT:
                out_copies.pop(b - NBUF_OUT).wait()

            src = in_v.at[b % NBUF_IN]
            dst = out_v.at[b % NBUF_OUT]

            @plsc.parallel_loop(0, BLK_ROWS, unroll=4)
            def body(r):
                rows = jnp.full((LANES,), r, jnp.int32)
                for t in range(K_OUT // LANES):
                    v = plsc.load_gather(src, [rows, cols[t]])
                    dst[r, pl.ds(LANES * t, LANES)] = v

            out_copies[b] = start_out(b)
            if b + NBUF_IN < NBLK:
                in_copies[b + NBUF_IN] = start_in(b + NBUF_IN)
        for b in sorted(out_copies):
            out_copies.pop(b).wait()

    return k


_SC_KERNEL = _build_sc_kernel()


def kernel(inputs):
    return _SC_KERNEL(inputs)
